# Initial kernel scaffold; baseline (speedup 1.0000x reference)
import jax
import jax.numpy as jnp
from jax.experimental import pallas as pl


def kernel(x, edge_index, edge_weight, W1, b1, W2, b2):
    raise NotImplementedError("write your pallas kernel here")


# speedup vs baseline: 10.8082x; 10.8082x over previous
import functools

import jax
import jax.numpy as jnp
from jax import lax
from jax.experimental import pallas as pl
from jax.experimental.pallas import tpu as pltpu
from jax.experimental.pallas import tpu_sc as plsc

N = 10000
E = 320000
NC = 2
NS = 16
NW = NC * NS
CH = 64
NCHUNK = E // CH
FULL = NCHUNK // NW
EXTRA = NCHUNK - FULL * NW
NPAD = 10240
ROWS_PER_TILE = NPAD // NS
ZR = 64


def _mm2_body(x_ref, wa_ref, wb_ref, xa_ref, y_ref):
    xv = x_ref[...]
    xa_ref[...] = jnp.dot(xv, wa_ref[...], preferred_element_type=jnp.float32)
    y_ref[...] = jnp.dot(xv, wb_ref[...], preferred_element_type=jnp.float32)


def _tc_mm2(x, wa, wb):
    return pl.pallas_call(
        _mm2_body,
        out_shape=(
            jax.ShapeDtypeStruct((x.shape[0], wa.shape[1]), jnp.float32),
            jax.ShapeDtypeStruct((x.shape[0], wb.shape[1]), jnp.float32),
        ),
    )(x, wa, wb)


def _fuse_body(xa_ref, p_ref, b_ref, wa_ref, h_ref, ha_ref):
    h = xa_ref[...] + p_ref[0] + p_ref[1] + b_ref[...]
    h = jnp.maximum(h, 0.0)
    h_ref[...] = h
    ha_ref[...] = jnp.dot(h, wa_ref[...], preferred_element_type=jnp.float32)


def _tc_fuse(xa, p, b, wa):
    return pl.pallas_call(
        _fuse_body,
        out_shape=(
            jax.ShapeDtypeStruct(xa.shape, jnp.float32),
            jax.ShapeDtypeStruct((xa.shape[0], wa.shape[1]), jnp.float32),
        ),
    )(xa, p, b.reshape(1, -1), wa)


def _final_body(ha_ref, p_ref, wb_ref, b_ref, o_ref):
    psum = p_ref[0] + p_ref[1]
    o_ref[...] = (ha_ref[...] + b_ref[...]
                  + jnp.dot(psum, wb_ref[...], preferred_element_type=jnp.float32))


def _tc_final(ha, p, wb, b):
    return pl.pallas_call(
        _final_body,
        out_shape=jax.ShapeDtypeStruct(ha.shape, jnp.float32),
    )(ha, p, wb, b.reshape(1, -1))


def _sc_spmm(y, src, dst, w):
    D = y.shape[1]
    mesh = plsc.VectorSubcoreMesh(core_axis_name="c", subcore_axis_name="s")

    @functools.partial(
        pl.kernel,
        out_type=jax.ShapeDtypeStruct((NC, NPAD, D), jnp.float32),
        mesh=mesh,
        scratch_types=[
            pltpu.VMEM_SHARED((NPAD, D), jnp.float32),
            pltpu.VMEM((4, CH), jnp.int32),
            pltpu.VMEM((4, CH), jnp.int32),
            pltpu.VMEM((4, CH), jnp.float32),
            pltpu.VMEM((CH, D), jnp.float32),
            pltpu.VMEM((CH, D), jnp.float32),
            pltpu.VMEM((CH, D), jnp.float32),
            pltpu.VMEM((CH, D), jnp.float32),
            pltpu.VMEM((ZR, D), jnp.float32),
            [pltpu.SemaphoreType.DMA] * 4,
            [pltpu.SemaphoreType.DMA] * 4,
            [pltpu.SemaphoreType.DMA] * 4,
        ],
    )
    def spmm(y_hbm, src_hbm, dst_hbm, w_hbm, out_hbm, acc, srcv, dstv, wv,
             r0b, r1b, r2b, r3b, zbuf, semG, semS, semI):
        c = lax.axis_index("c")
        s = lax.axis_index("s")
        wid = c * NS + s
        R = [r0b, r1b, r2b, r3b]

        zero = jnp.zeros((16,), jnp.float32)

        def zfill(i, _):
            zbuf[i // (D // 16), pl.ds((i % (D // 16)) * 16, 16)] = zero
            return 0

        lax.fori_loop(0, ZR * (D // 16), zfill, 0)
        row0 = s * ROWS_PER_TILE

        def zcopy(j, _):
            pltpu.sync_copy(zbuf, acc.at[pl.ds(row0 + j * ZR, ZR)])
            return 0

        lax.fori_loop(0, ROWS_PER_TILE // ZR, zcopy, 0)
        plsc.subcore_barrier()

        def off_of(j):
            jc = jnp.minimum(j, FULL)
            return jnp.minimum(wid + NW * jc, NCHUNK - 1) * CH

        def idx_issue(slot, j):
            o = off_of(j)
            pltpu.async_copy(src_hbm.at[pl.ds(o, CH)], srcv.at[slot], semI[slot])
            pltpu.async_copy(dst_hbm.at[pl.ds(o, CH)], dstv.at[slot], semI[slot])
            pltpu.async_copy(w_hbm.at[pl.ds(o, CH)], wv.at[slot], semI[slot])

        def idx_wait(slot):
            pltpu.make_async_copy(src_hbm.at[pl.ds(0, CH)], srcv.at[slot], semI[slot]).wait()
            pltpu.make_async_copy(dst_hbm.at[pl.ds(0, CH)], dstv.at[slot], semI[slot]).wait()
            pltpu.make_async_copy(w_hbm.at[pl.ds(0, CH)], wv.at[slot], semI[slot]).wait()

        def g_issue(slot):
            pltpu.async_copy(y_hbm.at[srcv.at[slot]], R[slot], semG[slot])

        def g_wait(slot):
            pltpu.make_async_copy(y_hbm.at[srcv.at[slot]], R[slot], semG[slot]).wait()

        def scale(slot):
            rows = R[slot]

            @plsc.parallel_loop(0, CH // 16, unroll=2)
            def _(g):
                wvec = wv[slot, pl.ds(g * 16, 16)]
                for l in range(16):
                    wsc = wvec[l]
                    for f in range(D // 16):
                        i = g * 16 + l
                        rows[i, pl.ds(f * 16, 16)] = rows[i, pl.ds(f * 16, 16)] * wsc

        def s_issue(slot):
            pltpu.async_copy(R[slot], acc.at[dstv.at[slot]], semS[slot], add=True)

        def s_wait(slot):
            pltpu.make_async_copy(R[slot], acc.at[dstv.at[slot]], semS[slot]).wait()

        def turn(k, j, skip_iw=False, skip_sw=False):
            if not skip_iw:
                idx_wait((k + 1) % 4)
            g_issue((k + 1) % 4)
            if not skip_sw:
                s_wait((k + 2) % 4)
            idx_issue((k + 2) % 4, j + 2)
            g_wait(k)
            scale(k)
            s_issue(k)

        pltpu.sync_copy(src_hbm.at[pl.ds(off_of(0), CH)], srcv.at[0])
        pltpu.sync_copy(dst_hbm.at[pl.ds(off_of(0), CH)], dstv.at[0])
        pltpu.sync_copy(w_hbm.at[pl.ds(off_of(0), CH)], wv.at[0])
        pltpu.sync_copy(src_hbm.at[pl.ds(off_of(1), CH)], srcv.at[1])
        pltpu.sync_copy(dst_hbm.at[pl.ds(off_of(1), CH)], dstv.at[1])
        pltpu.sync_copy(w_hbm.at[pl.ds(off_of(1), CH)], wv.at[1])
        g_issue(0)
        turn(0, 0, skip_iw=True, skip_sw=True)
        turn(1, 1, skip_sw=True)
        turn(2, 2)
        turn(3, 3)

        def body4(g, _):
            j = 4 * g + 4
            turn(0, j, False)
            turn(1, j + 1, False)
            turn(2, j + 2, False)
            turn(3, j + 3, False)
            return 0

        lax.fori_loop(0, (FULL - 4) // 4, body4, 0)

        s_wait(2)
        s_wait(3)
        g_wait(0)
        idx_wait(1)

        @pl.when(wid < EXTRA)
        def _():
            scale(0)
            pltpu.sync_copy(R[0], acc.at[dstv.at[0]], add=True)

        plsc.subcore_barrier()
        pltpu.sync_copy(acc.at[pl.ds(row0, ROWS_PER_TILE)],
                        out_hbm.at[c, pl.ds(row0, ROWS_PER_TILE)])

    return spmm(y, src, dst, w)[:, :N]


def kernel(x, edge_index, edge_weight, W1, b1, W2, b2):
    dst = edge_index[0]
    src = edge_index[1]
    W1a, W1b = W1[0::2], W1[1::2]
    W2a, W2b = W2[0::2], W2[1::2]

    xa, y1 = _tc_mm2(x, W1a, W1b)
    p1 = _sc_spmm(y1, src, dst, edge_weight)
    h, ha = _tc_fuse(xa, p1, b1, W2a)
    p2 = _sc_spmm(h, src, dst, edge_weight)
    return _tc_final(ha, p2, W2b, b2)


# speedup vs baseline: 11.6282x; 1.0759x over previous
import functools

import jax
import jax.numpy as jnp
from jax import lax
from jax.experimental import pallas as pl
from jax.experimental.pallas import tpu as pltpu
from jax.experimental.pallas import tpu_sc as plsc

N = 10000
E = 320000
NC = 2
NS = 16
NW = NC * NS
CH = 64
NCHUNK = E // CH
FULL = NCHUNK // NW
EXTRA = NCHUNK - FULL * NW
NPAD = 10240
ROWS_PER_TILE = NPAD // NS
ZR = 64


def _mm2_body(x_ref, wa_ref, wb_ref, xa_ref, y_ref):
    xv = x_ref[...]
    xa_ref[...] = jnp.dot(xv, wa_ref[...], preferred_element_type=jnp.float32)
    y_ref[...] = jnp.dot(xv, wb_ref[...], preferred_element_type=jnp.float32)


def _tc_mm2(x, wa, wb):
    return pl.pallas_call(
        _mm2_body,
        out_shape=(
            jax.ShapeDtypeStruct((x.shape[0], wa.shape[1]), jnp.float32),
            jax.ShapeDtypeStruct((x.shape[0], wb.shape[1]), jnp.float32),
        ),
    )(x, wa, wb)


def _fuse_body(xa_ref, p_ref, b_ref, wa_ref, wb_ref, ha_ref, y2_ref):
    h = xa_ref[...] + p_ref[0] + p_ref[1] + b_ref[...]
    h = jnp.maximum(h, 0.0)
    ha_ref[...] = jnp.dot(h, wa_ref[...], preferred_element_type=jnp.float32)
    y2_ref[...] = jnp.dot(h, wb_ref[...], preferred_element_type=jnp.float32)


def _tc_fuse(xa, p, b, wa, wb):
    return pl.pallas_call(
        _fuse_body,
        out_shape=(
            jax.ShapeDtypeStruct((xa.shape[0], wa.shape[1]), jnp.float32),
            jax.ShapeDtypeStruct((xa.shape[0], wb.shape[1]), jnp.float32),
        ),
    )(xa, p, b.reshape(1, -1), wa, wb)


def _final_body(ha_ref, p_ref, b_ref, o_ref):
    o_ref[...] = ha_ref[...] + p_ref[0] + p_ref[1] + b_ref[...]


def _tc_final(ha, p, b):
    return pl.pallas_call(
        _final_body,
        out_shape=jax.ShapeDtypeStruct(ha.shape, jnp.float32),
    )(ha, p, b.reshape(1, -1))


def _sc_spmm(y, src, dst, w):
    D = y.shape[1]
    mesh = plsc.VectorSubcoreMesh(core_axis_name="c", subcore_axis_name="s")
    params = None
    if D % 128 != 0:
        params = pltpu.CompilerParams(use_tc_tiling_on_sc=False)

    @functools.partial(
        pl.kernel,
        out_type=jax.ShapeDtypeStruct((NC, NPAD, D), jnp.float32),
        mesh=mesh,
        compiler_params=params,
        scratch_types=[
            pltpu.VMEM_SHARED((NPAD, D), jnp.float32),
            pltpu.VMEM((4, CH), jnp.int32),
            pltpu.VMEM((4, CH), jnp.int32),
            pltpu.VMEM((4, CH), jnp.float32),
            pltpu.VMEM((CH, D), jnp.float32),
            pltpu.VMEM((CH, D), jnp.float32),
            pltpu.VMEM((CH, D), jnp.float32),
            pltpu.VMEM((CH, D), jnp.float32),
            pltpu.VMEM((ZR, D), jnp.float32),
            [pltpu.SemaphoreType.DMA] * 4,
            [pltpu.SemaphoreType.DMA] * 4,
            [pltpu.SemaphoreType.DMA] * 4,
        ],
    )
    def spmm(y_hbm, src_hbm, dst_hbm, w_hbm, out_hbm, acc, srcv, dstv, wv,
             r0b, r1b, r2b, r3b, zbuf, semG, semS, semI):
        c = lax.axis_index("c")
        s = lax.axis_index("s")
        wid = c * NS + s
        R = [r0b, r1b, r2b, r3b]

        zero = jnp.zeros((16,), jnp.float32)

        def zfill(i, _):
            zbuf[i // (D // 16), pl.ds((i % (D // 16)) * 16, 16)] = zero
            return 0

        lax.fori_loop(0, ZR * (D // 16), zfill, 0)
        row0 = s * ROWS_PER_TILE

        def zcopy(j, _):
            pltpu.sync_copy(zbuf, acc.at[pl.ds(row0 + j * ZR, ZR)])
            return 0

        lax.fori_loop(0, ROWS_PER_TILE // ZR, zcopy, 0)
        plsc.subcore_barrier()

        def off_of(j):
            jc = jnp.minimum(j, FULL)
            return jnp.minimum(wid + NW * jc, NCHUNK - 1) * CH

        def idx_issue(slot, j):
            o = off_of(j)
            pltpu.async_copy(src_hbm.at[pl.ds(o, CH)], srcv.at[slot], semI[slot])
            pltpu.async_copy(dst_hbm.at[pl.ds(o, CH)], dstv.at[slot], semI[slot])
            pltpu.async_copy(w_hbm.at[pl.ds(o, CH)], wv.at[slot], semI[slot])

        def idx_wait(slot):
            pltpu.make_async_copy(src_hbm.at[pl.ds(0, CH)], srcv.at[slot], semI[slot]).wait()
            pltpu.make_async_copy(dst_hbm.at[pl.ds(0, CH)], dstv.at[slot], semI[slot]).wait()
            pltpu.make_async_copy(w_hbm.at[pl.ds(0, CH)], wv.at[slot], semI[slot]).wait()

        def g_issue(slot):
            pltpu.async_copy(y_hbm.at[srcv.at[slot]], R[slot], semG[slot])

        def g_wait(slot):
            pltpu.make_async_copy(y_hbm.at[srcv.at[slot]], R[slot], semG[slot]).wait()

        def scale(slot):
            rows = R[slot]

            @plsc.parallel_loop(0, CH // 16, unroll=2)
            def _(g):
                wvec = wv[slot, pl.ds(g * 16, 16)]
                for l in range(16):
                    wsc = wvec[l]
                    for f in range(D // 16):
                        i = g * 16 + l
                        rows[i, pl.ds(f * 16, 16)] = rows[i, pl.ds(f * 16, 16)] * wsc

        def s_issue(slot):
            pltpu.async_copy(R[slot], acc.at[dstv.at[slot]], semS[slot], add=True)

        def s_wait(slot):
            pltpu.make_async_copy(R[slot], acc.at[dstv.at[slot]], semS[slot]).wait()

        def turn(k, j, skip_iw=False, skip_sw=False):
            if not skip_iw:
                idx_wait((k + 1) % 4)
            g_issue((k + 1) % 4)
            if not skip_sw:
                s_wait((k + 2) % 4)
            idx_issue((k + 2) % 4, j + 2)
            g_wait(k)
            scale(k)
            s_issue(k)

        pltpu.sync_copy(src_hbm.at[pl.ds(off_of(0), CH)], srcv.at[0])
        pltpu.sync_copy(dst_hbm.at[pl.ds(off_of(0), CH)], dstv.at[0])
        pltpu.sync_copy(w_hbm.at[pl.ds(off_of(0), CH)], wv.at[0])
        pltpu.sync_copy(src_hbm.at[pl.ds(off_of(1), CH)], srcv.at[1])
        pltpu.sync_copy(dst_hbm.at[pl.ds(off_of(1), CH)], dstv.at[1])
        pltpu.sync_copy(w_hbm.at[pl.ds(off_of(1), CH)], wv.at[1])
        g_issue(0)
        turn(0, 0, skip_iw=True, skip_sw=True)
        turn(1, 1, skip_sw=True)
        turn(2, 2)
        turn(3, 3)

        def body4(g, _):
            j = 4 * g + 4
            turn(0, j, False)
            turn(1, j + 1, False)
            turn(2, j + 2, False)
            turn(3, j + 3, False)
            return 0

        lax.fori_loop(0, (FULL - 4) // 4, body4, 0)

        s_wait(2)
        s_wait(3)
        g_wait(0)
        idx_wait(1)

        @pl.when(wid < EXTRA)
        def _():
            scale(0)
            pltpu.sync_copy(R[0], acc.at[dstv.at[0]], add=True)

        plsc.subcore_barrier()
        pltpu.sync_copy(acc.at[pl.ds(row0, ROWS_PER_TILE)],
                        out_hbm.at[c, pl.ds(row0, ROWS_PER_TILE)])

    return spmm(y, src, dst, w)[:, :N]


def kernel(x, edge_index, edge_weight, W1, b1, W2, b2):
    dst = edge_index[0]
    src = edge_index[1]
    W1a, W1b = W1[0::2], W1[1::2]
    W2a, W2b = W2[0::2], W2[1::2]

    xa, y1 = _tc_mm2(x, W1a, W1b)
    p1 = _sc_spmm(y1, src, dst, edge_weight)
    ha, y2 = _tc_fuse(xa, p1, b1, W2a, W2b)
    p2 = _sc_spmm(y2, src, dst, edge_weight)
    return _tc_final(ha, p2, b2)
